# baseline (speedup 1.0000x reference)
import functools
import math

import jax
import jax.numpy as jnp
from jax import lax
from jax.experimental import pallas as pl
from jax.experimental.pallas import tpu as pltpu

N_HEADS = 16
KT = 128

_C0 = 6.283203125
_C1 = -1.781781975296326e-05
_C2 = -6.608047442568932e-13
_INV_2PI = 0.15915494309189535


def _reduced_sincos(ang):
    k = jnp.floor(ang * _INV_2PI + 0.5)
    r = ((ang - k * _C0) - k * _C1) - k * _C2
    return jnp.sin(r), jnp.cos(r)


def _rope_emb(p8, dirs_ref, freqs_ref):
    f32 = jnp.float32
    px, py, pz = p8[:, 0:1], p8[:, 1:2], p8[:, 2:3]
    nrm = jnp.sqrt(px * px + py * py + pz * pz)
    inv = 1.0 / jnp.maximum(nrm, 1e-12)
    bf = jnp.bfloat16
    ux = (px * inv).astype(bf).astype(f32)
    uy = (py * inv).astype(bf).astype(f32)
    uz = (pz * inv).astype(bf).astype(f32)
    d0 = dirs_ref[0:1, :].astype(bf).astype(f32)
    d1 = dirs_ref[1:2, :].astype(bf).astype(f32)
    d2 = dirs_ref[2:3, :].astype(bf).astype(f32)
    ang = (ux * d0 + uy * d1 + uz * d2) * freqs_ref[...]
    sin_a, cos_a = _reduced_sincos(ang)
    return jnp.concatenate([sin_a, cos_a], axis=1)


def _compact_kernel(T, padc_ref, spikec_ref, ltri_ref,
                    gx_ref, cnt_ref, biasp_ref):
    f32 = jnp.float32
    N = padc_ref.shape[1]
    s_id = pl.program_id(0)
    sendc = ((spikec_ref[0] != 0) & (padc_ref[0] != 0)).astype(f32)
    cum = jnp.dot(ltri_ref[...], sendc, preferred_element_type=f32)
    n = jnp.sum(sendc, axis=0, keepdims=True)
    jrow = jax.lax.broadcasted_iota(jnp.int32, (1, N), 1).astype(f32)
    le = (cum <= jrow).astype(f32)
    idx = jnp.sum(le, axis=0, keepdims=True)
    idx = jnp.where(n == 0.0, jrow, idx)
    idx = jnp.minimum(idx, f32(N - 1)).astype(jnp.int32)
    gx_ref[0] = idx
    ni = n.astype(jnp.int32)
    cnt_ref[0] = ni
    jcol = jax.lax.broadcasted_iota(jnp.int32, (N, 1), 0)
    biasp_ref[0] = jnp.where(jcol < ni, f32(0), f32(-1e30))


def _prep_kernel(x_ref, pos_ref, rmsw_ref, wq_ref, dirs_ref, freqs_ref, q_ref):
    f32 = jnp.float32
    x = x_ref[0]
    D = x.shape[1]
    H = N_HEADS
    Dh = D // H

    var = jnp.mean(x * x, axis=1, keepdims=True)
    xn = x * jax.lax.rsqrt(var + 1e-6) * rmsw_ref[...]
    emb = _rope_emb(pos_ref[0], dirs_ref, freqs_ref)
    F2 = emb.shape[1]
    qk = jnp.concatenate([xn[:, 0:F2] + emb, xn[:, F2:]], axis=1)

    scale = 1.0 / math.sqrt(Dh)
    q = jnp.dot(qk, wq_ref[...], preferred_element_type=f32) * scale
    for h in range(H):
        q_ref[0, h] = q[:, h * Dh:(h + 1) * Dh]


def _packkv_kernel(idx_ref, cnt_ref, x_ref, pos_ref, rmsw_ref, wk_ref, wv_ref,
                   dirs_ref, freqs_ref, k_ref, v_ref, px_scr, pp_scr):
    f32 = jnp.float32
    N, D = x_ref.shape[1], x_ref.shape[2]
    H = N_HEADS
    Dh = D // H
    n = cnt_ref[0, 0, 0]
    n_eff = jnp.where(n == 0, N, n)
    nt = (n_eff + (KT - 1)) // KT

    def gather_body(j, _):
        i = idx_ref[0, 0, j]
        px_scr[pl.ds(j, 1), :] = x_ref[0, pl.ds(i, 1), :]
        pp_scr[pl.ds(j, 1), :] = pos_ref[0, pl.ds(i, 1), :]
        return 0

    jax.lax.fori_loop(0, n_eff, gather_body, 0)

    def zero_body(j, _):
        px_scr[pl.ds(j, 1), :] = jnp.zeros((1, D), f32)
        pp_scr[pl.ds(j, 1), :] = jnp.zeros((1, 3), f32)
        return 0

    jax.lax.fori_loop(n_eff, nt * KT, zero_body, 0)

    def proj_body(jt, _):
        r0 = jt * KT
        xr = px_scr[pl.ds(r0, KT), :]
        var = jnp.mean(xr * xr, axis=1, keepdims=True)
        xn = xr * jax.lax.rsqrt(var + 1e-6) * rmsw_ref[...]
        emb = _rope_emb(pp_scr[pl.ds(r0, KT), :], dirs_ref, freqs_ref)
        F2 = emb.shape[1]
        rows_k = jnp.concatenate([xn[:, 0:F2] + emb, xn[:, F2:]], axis=1)
        kt = jnp.dot(rows_k, wk_ref[...], preferred_element_type=f32)
        vt = jnp.dot(xn, wv_ref[...], preferred_element_type=f32)
        for h in range(H):
            sl = slice(h * Dh, (h + 1) * Dh)
            k_ref[0, h, pl.ds(r0, KT), :] = kt[:, sl]
            v_ref[0, h, pl.ds(r0, KT), :] = vt[:, sl]
        return 0

    jax.lax.fori_loop(0, nt, proj_body, 0)


def _attn_kernel(cnt_ref, q_ref, k_ref, v_ref, biasp_ref, padc_ref, o_ref):
    f32 = jnp.float32
    q = q_ref[0, 0]
    N, Dh = q.shape
    n = cnt_ref[0, 0, 0]
    n_eff = jnp.where(n == 0, N, n)
    nt = (n_eff + (KT - 1)) // KT

    keepc = (padc_ref[0] != 0).astype(f32)
    ones = jnp.ones((N, 1), f32)
    q_aug = jnp.concatenate([q, ones], axis=1)

    m0 = jnp.full((N, 1), -3e38, f32)
    l0 = jnp.zeros((N, 1), f32)
    acc0 = jnp.zeros((N, Dh), f32)

    def tile_body(jt, carry):
        m, l, acc = carry
        r0 = jt * KT
        k_t = k_ref[0, 0, pl.ds(r0, KT), :]
        v_t = v_ref[0, 0, pl.ds(r0, KT), :]
        b_t = biasp_ref[0, pl.ds(r0, KT), :]
        k_aug = jnp.concatenate([k_t, b_t], axis=1)
        s_t = jax.lax.dot_general(q_aug, k_aug, (((1,), (1,)), ((), ())),
                                  preferred_element_type=f32)
        m_t = jnp.max(s_t, axis=1, keepdims=True)
        m_new = jnp.maximum(m, m_t)
        alpha = jnp.exp(m - m_new)
        e_t = jnp.exp(s_t - m_new)
        l_new = l * alpha + jnp.sum(e_t, axis=1, keepdims=True)
        acc_new = acc * alpha + jnp.dot(e_t, v_t, preferred_element_type=f32)
        return m_new, l_new, acc_new

    m, l, acc = jax.lax.fori_loop(0, nt, tile_body, (m0, l0, acc0))
    o_ref[0, 0] = acc * (keepc / l)


def _proj_kernel(att_ref, x_ref, wo_ref, o_ref):
    f32 = jnp.float32
    H = att_ref.shape[1]
    out = jnp.concatenate([att_ref[0, h] for h in range(H)], axis=1)
    o_ref[0] = x_ref[0] + jnp.dot(out, wo_ref[...], preferred_element_type=f32)


def kernel(x, point_positions, neuron_pad_mask, spike_mask, rms_w,
           Wq, Wk, Wv, Wo, rope_dirs, rope_freqs):
    B, T, N, D = x.shape
    S = B * T
    H = N_HEADS
    Dh = D // H
    F = rope_dirs.shape[0]
    F2 = 2 * F
    Rb = 256
    R = N // Rb
    f32 = jnp.float32
    i32 = jnp.int32

    xs = x.reshape(S, N, D)
    padc = neuron_pad_mask.reshape(B, N, 1)
    spikec = spike_mask.reshape(S, N, 1)
    rmsw2 = rms_w.reshape(1, D)
    dirs_t = rope_dirs.T
    freqs2 = rope_freqs.reshape(1, F)
    wq_t, wk_t, wv_t, wo_t = Wq.T, Wk.T, Wv.T, Wo.T
    iota_r = jax.lax.broadcasted_iota(f32, (N, N), 0)
    ltri = (jax.lax.broadcasted_iota(f32, (N, N), 1) <= iota_r).astype(f32)

    qkv_shape = jax.ShapeDtypeStruct((S, H, N, Dh), f32)
    c2 = lambda *_: (0, 0)

    sidx, scnt, biasp = pl.pallas_call(
        functools.partial(_compact_kernel, T),
        grid=(S,),
        in_specs=[
            pl.BlockSpec((1, N, 1), lambda s: (s // T, 0, 0)),
            pl.BlockSpec((1, N, 1), lambda s: (s, 0, 0)),
            pl.BlockSpec((N, N), lambda s: (0, 0)),
        ],
        out_specs=[
            pl.BlockSpec((1, 1, N), lambda s: (s, 0, 0)),
            pl.BlockSpec((1, 1, 1), lambda s: (s, 0, 0)),
            pl.BlockSpec((1, N, 1), lambda s: (s, 0, 0)),
        ],
        out_shape=[jax.ShapeDtypeStruct((S, 1, N), i32),
                   jax.ShapeDtypeStruct((S, 1, 1), i32),
                   jax.ShapeDtypeStruct((S, N, 1), f32)],
    )(padc, spikec, ltri)

    q4 = pl.pallas_call(
        _prep_kernel,
        grid=(S, R),
        in_specs=[
            pl.BlockSpec((1, Rb, D), lambda s, r: (s, r, 0)),
            pl.BlockSpec((1, Rb, 3), lambda s, r: (s // T, r, 0)),
            pl.BlockSpec((1, D), c2),
            pl.BlockSpec((D, D), c2),
            pl.BlockSpec((3, F), c2),
            pl.BlockSpec((1, F), c2),
        ],
        out_specs=pl.BlockSpec((1, H, Rb, Dh), lambda s, r: (s, 0, r, 0)),
        out_shape=qkv_shape,
        compiler_params=pltpu.CompilerParams(
            dimension_semantics=("parallel", "parallel")),
    )(xs, point_positions, rmsw2, wq_t, dirs_t, freqs2)

    k4, v4 = pl.pallas_call(
        _packkv_kernel,
        grid=(S,),
        in_specs=[
            pl.BlockSpec(memory_space=pltpu.SMEM,
                         block_shape=(1, 1, N), index_map=lambda s: (s, 0, 0)),
            pl.BlockSpec(memory_space=pltpu.SMEM,
                         block_shape=(1, 1, 1), index_map=lambda s: (s, 0, 0)),
            pl.BlockSpec((1, N, D), lambda s: (s, 0, 0)),
            pl.BlockSpec((1, N, 3), lambda s: (s // T, 0, 0)),
            pl.BlockSpec((1, D), lambda s: (0, 0)),
            pl.BlockSpec((D, D), lambda s: (0, 0)),
            pl.BlockSpec((D, D), lambda s: (0, 0)),
            pl.BlockSpec((3, F), lambda s: (0, 0)),
            pl.BlockSpec((1, F), lambda s: (0, 0)),
        ],
        out_specs=[
            pl.BlockSpec((1, H, N, Dh), lambda s: (s, 0, 0, 0)),
            pl.BlockSpec((1, H, N, Dh), lambda s: (s, 0, 0, 0)),
        ],
        out_shape=[qkv_shape, qkv_shape],
        scratch_shapes=[pltpu.VMEM((N, D), jnp.float32),
                        pltpu.VMEM((N, 3), jnp.float32)],
    )(sidx, scnt, xs, point_positions, rmsw2, wk_t, wv_t, dirs_t, freqs2)

    att = pl.pallas_call(
        _attn_kernel,
        grid=(S, H),
        in_specs=[
            pl.BlockSpec(memory_space=pltpu.SMEM,
                         block_shape=(1, 1, 1), index_map=lambda s, h: (s, 0, 0)),
            pl.BlockSpec((1, 1, N, Dh), lambda s, h: (s, h, 0, 0)),
            pl.BlockSpec((1, 1, N, Dh), lambda s, h: (s, h, 0, 0)),
            pl.BlockSpec((1, 1, N, Dh), lambda s, h: (s, h, 0, 0)),
            pl.BlockSpec((1, N, 1), lambda s, h: (s, 0, 0)),
            pl.BlockSpec((1, N, 1), lambda s, h: (s // T, 0, 0)),
        ],
        out_specs=pl.BlockSpec((1, 1, N, Dh), lambda s, h: (s, h, 0, 0)),
        out_shape=qkv_shape,
        compiler_params=pltpu.CompilerParams(
            dimension_semantics=("parallel", "parallel")),
    )(scnt, q4, k4, v4, biasp, padc)

    o = pl.pallas_call(
        _proj_kernel,
        grid=(S, R),
        in_specs=[
            pl.BlockSpec((1, H, Rb, Dh), lambda s, r: (s, 0, r, 0)),
            pl.BlockSpec((1, Rb, D), lambda s, r: (s, r, 0)),
            pl.BlockSpec((D, D), c2),
        ],
        out_specs=pl.BlockSpec((1, Rb, D), lambda s, r: (s, r, 0)),
        out_shape=jax.ShapeDtypeStruct((S, N, D), f32),
        compiler_params=pltpu.CompilerParams(
            dimension_semantics=("parallel", "parallel")),
    )(att, xs, wo_t)

    return o.reshape(B, T, N, D)


# speedup vs baseline: 1.0891x; 1.0891x over previous
import functools
import math

import jax
import jax.numpy as jnp
from jax import lax
from jax.experimental import pallas as pl
from jax.experimental.pallas import tpu as pltpu

N_HEADS = 16
KT = 512

_C0 = 6.283203125
_C1 = -1.781781975296326e-05
_C2 = -6.608047442568932e-13
_INV_2PI = 0.15915494309189535


def _reduced_sincos(ang):
    k = jnp.floor(ang * _INV_2PI + 0.5)
    r = ((ang - k * _C0) - k * _C1) - k * _C2
    return jnp.sin(r), jnp.cos(r)


def _rope_emb(p8, dirs_ref, freqs_ref):
    f32 = jnp.float32
    px, py, pz = p8[:, 0:1], p8[:, 1:2], p8[:, 2:3]
    nrm = jnp.sqrt(px * px + py * py + pz * pz)
    inv = 1.0 / jnp.maximum(nrm, 1e-12)
    bf = jnp.bfloat16
    ux = (px * inv).astype(bf).astype(f32)
    uy = (py * inv).astype(bf).astype(f32)
    uz = (pz * inv).astype(bf).astype(f32)
    d0 = dirs_ref[0:1, :].astype(bf).astype(f32)
    d1 = dirs_ref[1:2, :].astype(bf).astype(f32)
    d2 = dirs_ref[2:3, :].astype(bf).astype(f32)
    ang = (ux * d0 + uy * d1 + uz * d2) * freqs_ref[...]
    sin_a, cos_a = _reduced_sincos(ang)
    return jnp.concatenate([sin_a, cos_a], axis=1)


def _compact_kernel(T, padc_ref, spikec_ref, ltri_ref,
                    gx_ref, cnt_ref, biasp_ref):
    f32 = jnp.float32
    N = padc_ref.shape[1]
    s_id = pl.program_id(0)
    sendc = ((spikec_ref[0] != 0) & (padc_ref[0] != 0)).astype(f32)
    cum = jnp.dot(ltri_ref[...], sendc, preferred_element_type=f32)
    n = jnp.sum(sendc, axis=0, keepdims=True)
    jrow = jax.lax.broadcasted_iota(jnp.int32, (1, N), 1).astype(f32)
    le = (cum <= jrow).astype(f32)
    idx = jnp.sum(le, axis=0, keepdims=True)
    idx = jnp.where(n == 0.0, jrow, idx)
    idx = jnp.minimum(idx, f32(N - 1)).astype(jnp.int32)
    gx_ref[0] = idx
    ni = n.astype(jnp.int32)
    cnt_ref[0] = ni
    jcol = jax.lax.broadcasted_iota(jnp.int32, (N, 1), 0)
    biasp_ref[0] = jnp.where(jcol < ni, f32(0), f32(-1e30))


def _prep_kernel(x_ref, pos_ref, rmsw_ref, wq_ref, dirs_ref, freqs_ref, q_ref):
    f32 = jnp.float32
    x = x_ref[0]
    D = x.shape[1]
    H = N_HEADS
    Dh = D // H

    var = jnp.mean(x * x, axis=1, keepdims=True)
    xn = x * jax.lax.rsqrt(var + 1e-6) * rmsw_ref[...]
    emb = _rope_emb(pos_ref[0], dirs_ref, freqs_ref)
    F2 = emb.shape[1]
    qk = jnp.concatenate([xn[:, 0:F2] + emb, xn[:, F2:]], axis=1)

    scale = 1.0 / math.sqrt(Dh)
    q = jnp.dot(qk, wq_ref[...], preferred_element_type=f32) * scale
    for h in range(H):
        q_ref[0, h] = q[:, h * Dh:(h + 1) * Dh]


def _packkv_kernel(idx_ref, cnt_ref, x_ref, pos_ref, rmsw_ref, wk_ref, wv_ref,
                   dirs_ref, freqs_ref, k_ref, v_ref, px_scr, pp_scr):
    f32 = jnp.float32
    N, D = x_ref.shape[1], x_ref.shape[2]
    H = N_HEADS
    Dh = D // H
    n = cnt_ref[0, 0, 0]
    n_eff = jnp.where(n == 0, N, n)
    nt = (n_eff + (KT - 1)) // KT

    def gather_body(j, _):
        i = idx_ref[0, 0, j]
        px_scr[pl.ds(j, 1), :] = x_ref[0, pl.ds(i, 1), :]
        pp_scr[pl.ds(j, 1), :] = pos_ref[0, pl.ds(i, 1), :]
        return 0

    jax.lax.fori_loop(0, n_eff, gather_body, 0)

    def zero_body(j, _):
        px_scr[pl.ds(j, 1), :] = jnp.zeros((1, D), f32)
        pp_scr[pl.ds(j, 1), :] = jnp.zeros((1, 3), f32)
        return 0

    jax.lax.fori_loop(n_eff, nt * KT, zero_body, 0)

    def proj_body(jt, _):
        r0 = jt * KT
        xr = px_scr[pl.ds(r0, KT), :]
        var = jnp.mean(xr * xr, axis=1, keepdims=True)
        xn = xr * jax.lax.rsqrt(var + 1e-6) * rmsw_ref[...]
        emb = _rope_emb(pp_scr[pl.ds(r0, KT), :], dirs_ref, freqs_ref)
        F2 = emb.shape[1]
        rows_k = jnp.concatenate([xn[:, 0:F2] + emb, xn[:, F2:]], axis=1)
        kt = jnp.dot(rows_k, wk_ref[...], preferred_element_type=f32)
        vt = jnp.dot(xn, wv_ref[...], preferred_element_type=f32)
        for h in range(H):
            sl = slice(h * Dh, (h + 1) * Dh)
            k_ref[0, h, pl.ds(r0, KT), :] = kt[:, sl]
            v_ref[0, h, pl.ds(r0, KT), :] = vt[:, sl]
        return 0

    jax.lax.fori_loop(0, nt, proj_body, 0)


def _attn_kernel(cnt_ref, q_ref, k_ref, v_ref, biasp_ref, padc_ref, o_ref):
    f32 = jnp.float32
    q = q_ref[0, 0]
    N, Dh = q.shape
    n = cnt_ref[0, 0, 0]
    n_eff = jnp.where(n == 0, N, n)
    nt = (n_eff + (KT - 1)) // KT

    keepc = (padc_ref[0] != 0).astype(f32)
    ones = jnp.ones((N, 1), f32)
    q_aug = jnp.concatenate([q, ones], axis=1)

    m0 = jnp.full((N, 1), -3e38, f32)
    l0 = jnp.zeros((N, 1), f32)
    acc0 = jnp.zeros((N, Dh), f32)

    def tile_body(jt, carry):
        m, l, acc = carry
        r0 = jt * KT
        k_t = k_ref[0, 0, pl.ds(r0, KT), :]
        v_t = v_ref[0, 0, pl.ds(r0, KT), :]
        b_t = biasp_ref[0, pl.ds(r0, KT), :]
        k_aug = jnp.concatenate([k_t, b_t], axis=1)
        s_t = jax.lax.dot_general(q_aug, k_aug, (((1,), (1,)), ((), ())),
                                  preferred_element_type=f32)
        m_t = jnp.max(s_t, axis=1, keepdims=True)
        m_new = jnp.maximum(m, m_t)
        alpha = jnp.exp(m - m_new)
        e_t = jnp.exp(s_t - m_new)
        l_new = l * alpha + jnp.sum(e_t, axis=1, keepdims=True)
        acc_new = acc * alpha + jnp.dot(e_t, v_t, preferred_element_type=f32)
        return m_new, l_new, acc_new

    m, l, acc = jax.lax.fori_loop(0, nt, tile_body, (m0, l0, acc0))
    o_ref[0, 0] = acc * (keepc / l)


def _proj_kernel(att_ref, x_ref, wo_ref, o_ref):
    f32 = jnp.float32
    H = att_ref.shape[1]
    out = jnp.concatenate([att_ref[0, h] for h in range(H)], axis=1)
    o_ref[0] = x_ref[0] + jnp.dot(out, wo_ref[...], preferred_element_type=f32)


def kernel(x, point_positions, neuron_pad_mask, spike_mask, rms_w,
           Wq, Wk, Wv, Wo, rope_dirs, rope_freqs):
    B, T, N, D = x.shape
    S = B * T
    H = N_HEADS
    Dh = D // H
    F = rope_dirs.shape[0]
    F2 = 2 * F
    Rb = 256
    R = N // Rb
    f32 = jnp.float32
    i32 = jnp.int32

    xs = x.reshape(S, N, D)
    padc = neuron_pad_mask.reshape(B, N, 1)
    spikec = spike_mask.reshape(S, N, 1)
    rmsw2 = rms_w.reshape(1, D)
    dirs_t = rope_dirs.T
    freqs2 = rope_freqs.reshape(1, F)
    wq_t, wk_t, wv_t, wo_t = Wq.T, Wk.T, Wv.T, Wo.T
    iota_r = jax.lax.broadcasted_iota(f32, (N, N), 0)
    ltri = (jax.lax.broadcasted_iota(f32, (N, N), 1) <= iota_r).astype(f32)

    qkv_shape = jax.ShapeDtypeStruct((S, H, N, Dh), f32)
    c2 = lambda *_: (0, 0)

    sidx, scnt, biasp = pl.pallas_call(
        functools.partial(_compact_kernel, T),
        grid=(S,),
        in_specs=[
            pl.BlockSpec((1, N, 1), lambda s: (s // T, 0, 0)),
            pl.BlockSpec((1, N, 1), lambda s: (s, 0, 0)),
            pl.BlockSpec((N, N), lambda s: (0, 0)),
        ],
        out_specs=[
            pl.BlockSpec((1, 1, N), lambda s: (s, 0, 0)),
            pl.BlockSpec((1, 1, 1), lambda s: (s, 0, 0)),
            pl.BlockSpec((1, N, 1), lambda s: (s, 0, 0)),
        ],
        out_shape=[jax.ShapeDtypeStruct((S, 1, N), i32),
                   jax.ShapeDtypeStruct((S, 1, 1), i32),
                   jax.ShapeDtypeStruct((S, N, 1), f32)],
    )(padc, spikec, ltri)

    q4 = pl.pallas_call(
        _prep_kernel,
        grid=(S, R),
        in_specs=[
            pl.BlockSpec((1, Rb, D), lambda s, r: (s, r, 0)),
            pl.BlockSpec((1, Rb, 3), lambda s, r: (s // T, r, 0)),
            pl.BlockSpec((1, D), c2),
            pl.BlockSpec((D, D), c2),
            pl.BlockSpec((3, F), c2),
            pl.BlockSpec((1, F), c2),
        ],
        out_specs=pl.BlockSpec((1, H, Rb, Dh), lambda s, r: (s, 0, r, 0)),
        out_shape=qkv_shape,
        compiler_params=pltpu.CompilerParams(
            dimension_semantics=("parallel", "parallel")),
    )(xs, point_positions, rmsw2, wq_t, dirs_t, freqs2)

    k4, v4 = pl.pallas_call(
        _packkv_kernel,
        grid=(S,),
        in_specs=[
            pl.BlockSpec(memory_space=pltpu.SMEM,
                         block_shape=(1, 1, N), index_map=lambda s: (s, 0, 0)),
            pl.BlockSpec(memory_space=pltpu.SMEM,
                         block_shape=(1, 1, 1), index_map=lambda s: (s, 0, 0)),
            pl.BlockSpec((1, N, D), lambda s: (s, 0, 0)),
            pl.BlockSpec((1, N, 3), lambda s: (s // T, 0, 0)),
            pl.BlockSpec((1, D), lambda s: (0, 0)),
            pl.BlockSpec((D, D), lambda s: (0, 0)),
            pl.BlockSpec((D, D), lambda s: (0, 0)),
            pl.BlockSpec((3, F), lambda s: (0, 0)),
            pl.BlockSpec((1, F), lambda s: (0, 0)),
        ],
        out_specs=[
            pl.BlockSpec((1, H, N, Dh), lambda s: (s, 0, 0, 0)),
            pl.BlockSpec((1, H, N, Dh), lambda s: (s, 0, 0, 0)),
        ],
        out_shape=[qkv_shape, qkv_shape],
        scratch_shapes=[pltpu.VMEM((N, D), jnp.float32),
                        pltpu.VMEM((N, 3), jnp.float32)],
    )(sidx, scnt, xs, point_positions, rmsw2, wk_t, wv_t, dirs_t, freqs2)

    att = pl.pallas_call(
        _attn_kernel,
        grid=(S, H),
        in_specs=[
            pl.BlockSpec(memory_space=pltpu.SMEM,
                         block_shape=(1, 1, 1), index_map=lambda s, h: (s, 0, 0)),
            pl.BlockSpec((1, 1, N, Dh), lambda s, h: (s, h, 0, 0)),
            pl.BlockSpec((1, 1, N, Dh), lambda s, h: (s, h, 0, 0)),
            pl.BlockSpec((1, 1, N, Dh), lambda s, h: (s, h, 0, 0)),
            pl.BlockSpec((1, N, 1), lambda s, h: (s, 0, 0)),
            pl.BlockSpec((1, N, 1), lambda s, h: (s // T, 0, 0)),
        ],
        out_specs=pl.BlockSpec((1, 1, N, Dh), lambda s, h: (s, h, 0, 0)),
        out_shape=qkv_shape,
        compiler_params=pltpu.CompilerParams(
            dimension_semantics=("parallel", "parallel")),
    )(scnt, q4, k4, v4, biasp, padc)

    o = pl.pallas_call(
        _proj_kernel,
        grid=(S, R),
        in_specs=[
            pl.BlockSpec((1, H, Rb, Dh), lambda s, r: (s, 0, r, 0)),
            pl.BlockSpec((1, Rb, D), lambda s, r: (s, r, 0)),
            pl.BlockSpec((D, D), c2),
        ],
        out_specs=pl.BlockSpec((1, Rb, D), lambda s, r: (s, r, 0)),
        out_shape=jax.ShapeDtypeStruct((S, N, D), f32),
        compiler_params=pltpu.CompilerParams(
            dimension_semantics=("parallel", "parallel")),
    )(att, xs, wo_t)

    return o.reshape(B, T, N, D)


# speedup vs baseline: 1.1352x; 1.0423x over previous
import functools
import math

import jax
import jax.numpy as jnp
from jax import lax
from jax.experimental import pallas as pl
from jax.experimental.pallas import tpu as pltpu

N_HEADS = 16
KT = 256

_C0 = 6.283203125
_C1 = -1.781781975296326e-05
_C2 = -6.608047442568932e-13
_INV_2PI = 0.15915494309189535


def _reduced_sincos(ang):
    k = jnp.floor(ang * _INV_2PI + 0.5)
    r = ((ang - k * _C0) - k * _C1) - k * _C2
    return jnp.sin(r), jnp.cos(r)


def _rope_emb(p8, dirs_ref, freqs_ref):
    f32 = jnp.float32
    px, py, pz = p8[:, 0:1], p8[:, 1:2], p8[:, 2:3]
    nrm = jnp.sqrt(px * px + py * py + pz * pz)
    inv = 1.0 / jnp.maximum(nrm, 1e-12)
    bf = jnp.bfloat16
    ux = (px * inv).astype(bf).astype(f32)
    uy = (py * inv).astype(bf).astype(f32)
    uz = (pz * inv).astype(bf).astype(f32)
    d0 = dirs_ref[0:1, :].astype(bf).astype(f32)
    d1 = dirs_ref[1:2, :].astype(bf).astype(f32)
    d2 = dirs_ref[2:3, :].astype(bf).astype(f32)
    ang = (ux * d0 + uy * d1 + uz * d2) * freqs_ref[...]
    sin_a, cos_a = _reduced_sincos(ang)
    return jnp.concatenate([sin_a, cos_a], axis=1)


def _compact_kernel(T, padc_ref, spikec_ref, ltri_ref,
                    gx_ref, cnt_ref, biasp_ref):
    f32 = jnp.float32
    N = padc_ref.shape[1]
    s_id = pl.program_id(0)
    sendc = ((spikec_ref[0] != 0) & (padc_ref[0] != 0)).astype(f32)
    cum = jnp.dot(ltri_ref[...], sendc, preferred_element_type=f32)
    n = jnp.sum(sendc, axis=0, keepdims=True)
    jrow = jax.lax.broadcasted_iota(jnp.int32, (1, N), 1).astype(f32)
    le = (cum <= jrow).astype(f32)
    idx = jnp.sum(le, axis=0, keepdims=True)
    idx = jnp.where(n == 0.0, jrow, idx)
    idx = jnp.minimum(idx, f32(N - 1)).astype(jnp.int32)
    gx_ref[0] = idx
    ni = n.astype(jnp.int32)
    cnt_ref[0] = ni
    jcol = jax.lax.broadcasted_iota(jnp.int32, (N, 1), 0)
    biasp_ref[0] = jnp.where(jcol < ni, f32(0), f32(-1e30))


def _prep_kernel(x_ref, pos_ref, rmsw_ref, wq_ref, dirs_ref, freqs_ref, q_ref):
    f32 = jnp.float32
    x = x_ref[0]
    D = x.shape[1]
    H = N_HEADS
    Dh = D // H

    var = jnp.mean(x * x, axis=1, keepdims=True)
    xn = x * jax.lax.rsqrt(var + 1e-6) * rmsw_ref[...]
    emb = _rope_emb(pos_ref[0], dirs_ref, freqs_ref)
    F2 = emb.shape[1]
    qk = jnp.concatenate([xn[:, 0:F2] + emb, xn[:, F2:]], axis=1)

    scale = 1.0 / math.sqrt(Dh)
    q = jnp.dot(qk, wq_ref[...], preferred_element_type=f32) * scale
    for h in range(H):
        q_ref[0, h] = q[:, h * Dh:(h + 1) * Dh]


def _packkv_kernel(idx_ref, cnt_ref, x_ref, pos_ref, rmsw_ref, wk_ref, wv_ref,
                   dirs_ref, freqs_ref, k_ref, v_ref, px_scr, pp_scr):
    f32 = jnp.float32
    N, D = x_ref.shape[1], x_ref.shape[2]
    H = N_HEADS
    Dh = D // H
    n = cnt_ref[0, 0, 0]
    n_eff = jnp.where(n == 0, N, n)
    nt = (n_eff + (KT - 1)) // KT

    def gather_body(j, _):
        i = idx_ref[0, 0, j]
        px_scr[pl.ds(j, 1), :] = x_ref[0, pl.ds(i, 1), :]
        pp_scr[pl.ds(j, 1), :] = pos_ref[0, pl.ds(i, 1), :]
        return 0

    jax.lax.fori_loop(0, n_eff, gather_body, 0)

    def zero_body(j, _):
        px_scr[pl.ds(j, 1), :] = jnp.zeros((1, D), f32)
        pp_scr[pl.ds(j, 1), :] = jnp.zeros((1, 3), f32)
        return 0

    jax.lax.fori_loop(n_eff, nt * KT, zero_body, 0)

    def proj_body(jt, _):
        r0 = jt * KT
        xr = px_scr[pl.ds(r0, KT), :]
        var = jnp.mean(xr * xr, axis=1, keepdims=True)
        xn = xr * jax.lax.rsqrt(var + 1e-6) * rmsw_ref[...]
        emb = _rope_emb(pp_scr[pl.ds(r0, KT), :], dirs_ref, freqs_ref)
        F2 = emb.shape[1]
        rows_k = jnp.concatenate([xn[:, 0:F2] + emb, xn[:, F2:]], axis=1)
        kt = jnp.dot(rows_k, wk_ref[...], preferred_element_type=f32)
        vt = jnp.dot(xn, wv_ref[...], preferred_element_type=f32)
        for h in range(H):
            sl = slice(h * Dh, (h + 1) * Dh)
            k_ref[0, h, pl.ds(r0, KT), :] = kt[:, sl]
            v_ref[0, h, pl.ds(r0, KT), :] = vt[:, sl]
        return 0

    jax.lax.fori_loop(0, nt, proj_body, 0)


def _attn_kernel(cnt_ref, q_ref, k_ref, v_ref, biasp_ref, padc_ref, o_ref):
    f32 = jnp.float32
    q = q_ref[0, 0]
    N, Dh = q.shape
    n = cnt_ref[0, 0, 0]
    n_eff = jnp.where(n == 0, N, n)
    nt = (n_eff + (KT - 1)) // KT

    keepc = (padc_ref[0] != 0).astype(f32)
    ones = jnp.ones((N, 1), f32)
    q_aug = jnp.concatenate([q, ones], axis=1)

    m0 = jnp.full((N, 1), -3e38, f32)
    l0 = jnp.zeros((N, 1), f32)
    acc0 = jnp.zeros((N, Dh), f32)

    def tile_body(jt, carry):
        m, l, acc = carry
        r0 = jt * KT
        k_t = k_ref[0, 0, pl.ds(r0, KT), :]
        v_t = v_ref[0, 0, pl.ds(r0, KT), :]
        b_t = biasp_ref[0, pl.ds(r0, KT), :]
        k_aug = jnp.concatenate([k_t, b_t], axis=1)
        s_t = jax.lax.dot_general(q_aug, k_aug, (((1,), (1,)), ((), ())),
                                  preferred_element_type=f32)
        m_t = jnp.max(s_t, axis=1, keepdims=True)
        m_new = jnp.maximum(m, m_t)
        alpha = jnp.exp(m - m_new)
        e_t = jnp.exp(s_t - m_new)
        l_new = l * alpha + jnp.sum(e_t, axis=1, keepdims=True)
        acc_new = acc * alpha + jnp.dot(e_t.astype(jnp.bfloat16), v_t.astype(jnp.bfloat16),
                                       preferred_element_type=f32)
        return m_new, l_new, acc_new

    m, l, acc = jax.lax.fori_loop(0, nt, tile_body, (m0, l0, acc0))
    o_ref[0, 0] = acc * (keepc / l)


def _proj_kernel(att_ref, x_ref, wo_ref, o_ref):
    f32 = jnp.float32
    H = att_ref.shape[1]
    out = jnp.concatenate([att_ref[0, h] for h in range(H)], axis=1)
    o_ref[0] = x_ref[0] + jnp.dot(out, wo_ref[...], preferred_element_type=f32)


def kernel(x, point_positions, neuron_pad_mask, spike_mask, rms_w,
           Wq, Wk, Wv, Wo, rope_dirs, rope_freqs):
    B, T, N, D = x.shape
    S = B * T
    H = N_HEADS
    Dh = D // H
    F = rope_dirs.shape[0]
    F2 = 2 * F
    Rb = 256
    R = N // Rb
    f32 = jnp.float32
    i32 = jnp.int32

    xs = x.reshape(S, N, D)
    padc = neuron_pad_mask.reshape(B, N, 1)
    spikec = spike_mask.reshape(S, N, 1)
    rmsw2 = rms_w.reshape(1, D)
    dirs_t = rope_dirs.T
    freqs2 = rope_freqs.reshape(1, F)
    wq_t, wk_t, wv_t, wo_t = Wq.T, Wk.T, Wv.T, Wo.T
    iota_r = jax.lax.broadcasted_iota(f32, (N, N), 0)
    ltri = (jax.lax.broadcasted_iota(f32, (N, N), 1) <= iota_r).astype(f32)

    qkv_shape = jax.ShapeDtypeStruct((S, H, N, Dh), f32)
    c2 = lambda *_: (0, 0)

    sidx, scnt, biasp = pl.pallas_call(
        functools.partial(_compact_kernel, T),
        grid=(S,),
        in_specs=[
            pl.BlockSpec((1, N, 1), lambda s: (s // T, 0, 0)),
            pl.BlockSpec((1, N, 1), lambda s: (s, 0, 0)),
            pl.BlockSpec((N, N), lambda s: (0, 0)),
        ],
        out_specs=[
            pl.BlockSpec((1, 1, N), lambda s: (s, 0, 0)),
            pl.BlockSpec((1, 1, 1), lambda s: (s, 0, 0)),
            pl.BlockSpec((1, N, 1), lambda s: (s, 0, 0)),
        ],
        out_shape=[jax.ShapeDtypeStruct((S, 1, N), i32),
                   jax.ShapeDtypeStruct((S, 1, 1), i32),
                   jax.ShapeDtypeStruct((S, N, 1), f32)],
    )(padc, spikec, ltri)

    q4 = pl.pallas_call(
        _prep_kernel,
        grid=(S, R),
        in_specs=[
            pl.BlockSpec((1, Rb, D), lambda s, r: (s, r, 0)),
            pl.BlockSpec((1, Rb, 3), lambda s, r: (s // T, r, 0)),
            pl.BlockSpec((1, D), c2),
            pl.BlockSpec((D, D), c2),
            pl.BlockSpec((3, F), c2),
            pl.BlockSpec((1, F), c2),
        ],
        out_specs=pl.BlockSpec((1, H, Rb, Dh), lambda s, r: (s, 0, r, 0)),
        out_shape=qkv_shape,
        compiler_params=pltpu.CompilerParams(
            dimension_semantics=("parallel", "parallel")),
    )(xs, point_positions, rmsw2, wq_t, dirs_t, freqs2)

    k4, v4 = pl.pallas_call(
        _packkv_kernel,
        grid=(S,),
        in_specs=[
            pl.BlockSpec(memory_space=pltpu.SMEM,
                         block_shape=(1, 1, N), index_map=lambda s: (s, 0, 0)),
            pl.BlockSpec(memory_space=pltpu.SMEM,
                         block_shape=(1, 1, 1), index_map=lambda s: (s, 0, 0)),
            pl.BlockSpec((1, N, D), lambda s: (s, 0, 0)),
            pl.BlockSpec((1, N, 3), lambda s: (s // T, 0, 0)),
            pl.BlockSpec((1, D), lambda s: (0, 0)),
            pl.BlockSpec((D, D), lambda s: (0, 0)),
            pl.BlockSpec((D, D), lambda s: (0, 0)),
            pl.BlockSpec((3, F), lambda s: (0, 0)),
            pl.BlockSpec((1, F), lambda s: (0, 0)),
        ],
        out_specs=[
            pl.BlockSpec((1, H, N, Dh), lambda s: (s, 0, 0, 0)),
            pl.BlockSpec((1, H, N, Dh), lambda s: (s, 0, 0, 0)),
        ],
        out_shape=[qkv_shape, qkv_shape],
        scratch_shapes=[pltpu.VMEM((N, D), jnp.float32),
                        pltpu.VMEM((N, 3), jnp.float32)],
    )(sidx, scnt, xs, point_positions, rmsw2, wk_t, wv_t, dirs_t, freqs2)

    att = pl.pallas_call(
        _attn_kernel,
        grid=(S, H),
        in_specs=[
            pl.BlockSpec(memory_space=pltpu.SMEM,
                         block_shape=(1, 1, 1), index_map=lambda s, h: (s, 0, 0)),
            pl.BlockSpec((1, 1, N, Dh), lambda s, h: (s, h, 0, 0)),
            pl.BlockSpec((1, 1, N, Dh), lambda s, h: (s, h, 0, 0)),
            pl.BlockSpec((1, 1, N, Dh), lambda s, h: (s, h, 0, 0)),
            pl.BlockSpec((1, N, 1), lambda s, h: (s, 0, 0)),
            pl.BlockSpec((1, N, 1), lambda s, h: (s // T, 0, 0)),
        ],
        out_specs=pl.BlockSpec((1, 1, N, Dh), lambda s, h: (s, h, 0, 0)),
        out_shape=qkv_shape,
        compiler_params=pltpu.CompilerParams(
            dimension_semantics=("parallel", "parallel")),
    )(scnt, q4, k4, v4, biasp, padc)

    o = pl.pallas_call(
        _proj_kernel,
        grid=(S, R),
        in_specs=[
            pl.BlockSpec((1, H, Rb, Dh), lambda s, r: (s, 0, r, 0)),
            pl.BlockSpec((1, Rb, D), lambda s, r: (s, r, 0)),
            pl.BlockSpec((D, D), c2),
        ],
        out_specs=pl.BlockSpec((1, Rb, D), lambda s, r: (s, r, 0)),
        out_shape=jax.ShapeDtypeStruct((S, N, D), f32),
        compiler_params=pltpu.CompilerParams(
            dimension_semantics=("parallel", "parallel")),
    )(att, xs, wo_t)

    return o.reshape(B, T, N, D)


# speedup vs baseline: 1.1777x; 1.0374x over previous
import functools
import math

import jax
import jax.numpy as jnp
from jax import lax
from jax.experimental import pallas as pl
from jax.experimental.pallas import tpu as pltpu

N_HEADS = 16
KT = 256

_C0 = 6.283203125
_C1 = -1.781781975296326e-05
_C2 = -6.608047442568932e-13
_INV_2PI = 0.15915494309189535


def _reduced_sincos(ang):
    k = jnp.floor(ang * _INV_2PI + 0.5)
    r = ((ang - k * _C0) - k * _C1) - k * _C2
    return jnp.sin(r), jnp.cos(r)


def _rope_emb(p8, dirs_ref, freqs_ref):
    f32 = jnp.float32
    px, py, pz = p8[:, 0:1], p8[:, 1:2], p8[:, 2:3]
    nrm = jnp.sqrt(px * px + py * py + pz * pz)
    inv = 1.0 / jnp.maximum(nrm, 1e-12)
    bf = jnp.bfloat16
    ux = (px * inv).astype(bf).astype(f32)
    uy = (py * inv).astype(bf).astype(f32)
    uz = (pz * inv).astype(bf).astype(f32)
    d0 = dirs_ref[0:1, :].astype(bf).astype(f32)
    d1 = dirs_ref[1:2, :].astype(bf).astype(f32)
    d2 = dirs_ref[2:3, :].astype(bf).astype(f32)
    ang = (ux * d0 + uy * d1 + uz * d2) * freqs_ref[...]
    sin_a, cos_a = _reduced_sincos(ang)
    return jnp.concatenate([sin_a, cos_a], axis=1)


def _compact_kernel(T, padc_ref, spikec_ref, ltri_ref,
                    gx_ref, cnt_ref, biasp_ref):
    f32 = jnp.float32
    N = padc_ref.shape[1]
    s_id = pl.program_id(0)
    sendc = ((spikec_ref[0] != 0) & (padc_ref[0] != 0)).astype(f32)
    cum = jnp.dot(ltri_ref[...], sendc, preferred_element_type=f32)
    n = jnp.sum(sendc, axis=0, keepdims=True)
    jrow = jax.lax.broadcasted_iota(jnp.int32, (1, N), 1).astype(f32)
    le = (cum <= jrow).astype(f32)
    idx = jnp.sum(le, axis=0, keepdims=True)
    idx = jnp.where(n == 0.0, jrow, idx)
    idx = jnp.minimum(idx, f32(N - 1)).astype(jnp.int32)
    gx_ref[0] = idx
    ni = n.astype(jnp.int32)
    cnt_ref[0] = ni
    jcol = jax.lax.broadcasted_iota(jnp.int32, (N, 1), 0)
    biasp_ref[0] = jnp.where(jcol < ni, f32(0), f32(-1e30))


def _prep_kernel(x_ref, pos_ref, rmsw_ref, wq_ref, dirs_ref, freqs_ref, q_ref):
    f32 = jnp.float32
    x = x_ref[0]
    D = x.shape[1]
    H = N_HEADS
    Dh = D // H

    var = jnp.mean(x * x, axis=1, keepdims=True)
    xn = x * jax.lax.rsqrt(var + 1e-6) * rmsw_ref[...]
    emb = _rope_emb(pos_ref[0], dirs_ref, freqs_ref)
    F2 = emb.shape[1]
    qk = jnp.concatenate([xn[:, 0:F2] + emb, xn[:, F2:]], axis=1)

    scale = 1.0 / math.sqrt(Dh)
    q = jnp.dot(qk, wq_ref[...], preferred_element_type=f32) * scale
    for h in range(H):
        q_ref[0, h] = q[:, h * Dh:(h + 1) * Dh]


def _packkv_kernel(idx_ref, cnt_ref, x_ref, pos_ref, rmsw_ref, wk_ref, wv_ref,
                   dirs_ref, freqs_ref, k_ref, v_ref, px_scr, pp_scr):
    f32 = jnp.float32
    N, D = x_ref.shape[1], x_ref.shape[2]
    H = N_HEADS
    Dh = D // H
    n = cnt_ref[0, 0, 0]
    n_eff = jnp.where(n == 0, N, n)
    nt = (n_eff + (KT - 1)) // KT

    def gather_body(j, _):
        i = idx_ref[0, 0, j]
        px_scr[pl.ds(j, 1), :] = x_ref[0, pl.ds(i, 1), :]
        pp_scr[pl.ds(j, 1), :] = pos_ref[0, pl.ds(i, 1), :]
        return 0

    jax.lax.fori_loop(0, n_eff, gather_body, 0)

    def zero_body(j, _):
        px_scr[pl.ds(j, 1), :] = jnp.zeros((1, D), f32)
        pp_scr[pl.ds(j, 1), :] = jnp.zeros((1, 3), f32)
        return 0

    jax.lax.fori_loop(n_eff, nt * KT, zero_body, 0)

    def proj_body(jt, _):
        r0 = jt * KT
        xr = px_scr[pl.ds(r0, KT), :]
        var = jnp.mean(xr * xr, axis=1, keepdims=True)
        xn = xr * jax.lax.rsqrt(var + 1e-6) * rmsw_ref[...]
        emb = _rope_emb(pp_scr[pl.ds(r0, KT), :], dirs_ref, freqs_ref)
        F2 = emb.shape[1]
        rows_k = jnp.concatenate([xn[:, 0:F2] + emb, xn[:, F2:]], axis=1)
        kt = jnp.dot(rows_k, wk_ref[...], preferred_element_type=f32)
        vt = jnp.dot(xn, wv_ref[...], preferred_element_type=f32)
        for h in range(H):
            sl = slice(h * Dh, (h + 1) * Dh)
            k_ref[0, h, pl.ds(r0, KT), :] = kt[:, sl]
            v_ref[0, h, pl.ds(r0, KT), :] = vt[:, sl]
        return 0

    jax.lax.fori_loop(0, nt, proj_body, 0)


def _attn_kernel(cnt_ref, q_ref, k_ref, v_ref, biasp_ref, padc_ref, o_ref):
    f32 = jnp.float32
    HB = q_ref.shape[1]
    N, Dh = q_ref.shape[2], q_ref.shape[3]
    n = cnt_ref[0, 0, 0]
    n_eff = jnp.where(n == 0, N, n)
    nt = (n_eff + (KT - 1)) // KT

    keepc = (padc_ref[0] != 0).astype(f32)
    ones = jnp.ones((N, 1), f32)

    for hh in range(HB):
        q_aug = jnp.concatenate([q_ref[0, hh], ones], axis=1)
        m0 = jnp.full((N, 1), -3e38, f32)
        l0 = jnp.zeros((N, 1), f32)
        acc0 = jnp.zeros((N, Dh), f32)

        def tile_body(jt, carry):
            m, l, acc = carry
            r0 = jt * KT
            k_t = k_ref[0, hh, pl.ds(r0, KT), :]
            v_t = v_ref[0, hh, pl.ds(r0, KT), :]
            b_t = biasp_ref[0, pl.ds(r0, KT), :]
            k_aug = jnp.concatenate([k_t, b_t], axis=1)
            s_t = jax.lax.dot_general(q_aug, k_aug, (((1,), (1,)), ((), ())),
                                      preferred_element_type=f32)
            m_t = jnp.max(s_t, axis=1, keepdims=True)
            m_new = jnp.maximum(m, m_t)
            alpha = jnp.exp(m - m_new)
            e_t = jnp.exp(s_t - m_new)
            l_new = l * alpha + jnp.sum(e_t, axis=1, keepdims=True)
            acc_new = acc * alpha + jnp.dot(e_t.astype(jnp.bfloat16),
                                            v_t.astype(jnp.bfloat16),
                                            preferred_element_type=f32)
            return m_new, l_new, acc_new

        m, l, acc = jax.lax.fori_loop(0, nt, tile_body, (m0, l0, acc0))
        o_ref[0, hh] = acc * (keepc / l)


def _proj_kernel(att_ref, x_ref, wo_ref, o_ref):
    f32 = jnp.float32
    H = att_ref.shape[1]
    out = jnp.concatenate([att_ref[0, h] for h in range(H)], axis=1)
    o_ref[0] = x_ref[0] + jnp.dot(out, wo_ref[...], preferred_element_type=f32)


def kernel(x, point_positions, neuron_pad_mask, spike_mask, rms_w,
           Wq, Wk, Wv, Wo, rope_dirs, rope_freqs):
    B, T, N, D = x.shape
    S = B * T
    H = N_HEADS
    Dh = D // H
    F = rope_dirs.shape[0]
    F2 = 2 * F
    Rb = 256
    R = N // Rb
    f32 = jnp.float32
    i32 = jnp.int32

    xs = x.reshape(S, N, D)
    padc = neuron_pad_mask.reshape(B, N, 1)
    spikec = spike_mask.reshape(S, N, 1)
    rmsw2 = rms_w.reshape(1, D)
    dirs_t = rope_dirs.T
    freqs2 = rope_freqs.reshape(1, F)
    wq_t, wk_t, wv_t, wo_t = Wq.T, Wk.T, Wv.T, Wo.T
    iota_r = jax.lax.broadcasted_iota(f32, (N, N), 0)
    ltri = (jax.lax.broadcasted_iota(f32, (N, N), 1) <= iota_r).astype(f32)

    qkv_shape = jax.ShapeDtypeStruct((S, H, N, Dh), f32)
    c2 = lambda *_: (0, 0)

    sidx, scnt, biasp = pl.pallas_call(
        functools.partial(_compact_kernel, T),
        grid=(S,),
        in_specs=[
            pl.BlockSpec((1, N, 1), lambda s: (s // T, 0, 0)),
            pl.BlockSpec((1, N, 1), lambda s: (s, 0, 0)),
            pl.BlockSpec((N, N), lambda s: (0, 0)),
        ],
        out_specs=[
            pl.BlockSpec((1, 1, N), lambda s: (s, 0, 0)),
            pl.BlockSpec((1, 1, 1), lambda s: (s, 0, 0)),
            pl.BlockSpec((1, N, 1), lambda s: (s, 0, 0)),
        ],
        out_shape=[jax.ShapeDtypeStruct((S, 1, N), i32),
                   jax.ShapeDtypeStruct((S, 1, 1), i32),
                   jax.ShapeDtypeStruct((S, N, 1), f32)],
    )(padc, spikec, ltri)

    q4 = pl.pallas_call(
        _prep_kernel,
        grid=(S, R),
        in_specs=[
            pl.BlockSpec((1, Rb, D), lambda s, r: (s, r, 0)),
            pl.BlockSpec((1, Rb, 3), lambda s, r: (s // T, r, 0)),
            pl.BlockSpec((1, D), c2),
            pl.BlockSpec((D, D), c2),
            pl.BlockSpec((3, F), c2),
            pl.BlockSpec((1, F), c2),
        ],
        out_specs=pl.BlockSpec((1, H, Rb, Dh), lambda s, r: (s, 0, r, 0)),
        out_shape=qkv_shape,
        compiler_params=pltpu.CompilerParams(
            dimension_semantics=("parallel", "parallel")),
    )(xs, point_positions, rmsw2, wq_t, dirs_t, freqs2)

    k4, v4 = pl.pallas_call(
        _packkv_kernel,
        grid=(S,),
        in_specs=[
            pl.BlockSpec(memory_space=pltpu.SMEM,
                         block_shape=(1, 1, N), index_map=lambda s: (s, 0, 0)),
            pl.BlockSpec(memory_space=pltpu.SMEM,
                         block_shape=(1, 1, 1), index_map=lambda s: (s, 0, 0)),
            pl.BlockSpec((1, N, D), lambda s: (s, 0, 0)),
            pl.BlockSpec((1, N, 3), lambda s: (s // T, 0, 0)),
            pl.BlockSpec((1, D), lambda s: (0, 0)),
            pl.BlockSpec((D, D), lambda s: (0, 0)),
            pl.BlockSpec((D, D), lambda s: (0, 0)),
            pl.BlockSpec((3, F), lambda s: (0, 0)),
            pl.BlockSpec((1, F), lambda s: (0, 0)),
        ],
        out_specs=[
            pl.BlockSpec((1, H, N, Dh), lambda s: (s, 0, 0, 0)),
            pl.BlockSpec((1, H, N, Dh), lambda s: (s, 0, 0, 0)),
        ],
        out_shape=[qkv_shape, qkv_shape],
        scratch_shapes=[pltpu.VMEM((N, D), jnp.float32),
                        pltpu.VMEM((N, 3), jnp.float32)],
    )(sidx, scnt, xs, point_positions, rmsw2, wk_t, wv_t, dirs_t, freqs2)

    HB = 2
    att = pl.pallas_call(
        _attn_kernel,
        grid=(S, H // HB),
        in_specs=[
            pl.BlockSpec(memory_space=pltpu.SMEM,
                         block_shape=(1, 1, 1), index_map=lambda s, h: (s, 0, 0)),
            pl.BlockSpec((1, HB, N, Dh), lambda s, h: (s, h, 0, 0)),
            pl.BlockSpec((1, HB, N, Dh), lambda s, h: (s, h, 0, 0)),
            pl.BlockSpec((1, HB, N, Dh), lambda s, h: (s, h, 0, 0)),
            pl.BlockSpec((1, N, 1), lambda s, h: (s, 0, 0)),
            pl.BlockSpec((1, N, 1), lambda s, h: (s // T, 0, 0)),
        ],
        out_specs=pl.BlockSpec((1, HB, N, Dh), lambda s, h: (s, h, 0, 0)),
        out_shape=qkv_shape,
        compiler_params=pltpu.CompilerParams(
            dimension_semantics=("parallel", "parallel")),
    )(scnt, q4, k4, v4, biasp, padc)

    o = pl.pallas_call(
        _proj_kernel,
        grid=(S, R),
        in_specs=[
            pl.BlockSpec((1, H, Rb, Dh), lambda s, r: (s, 0, r, 0)),
            pl.BlockSpec((1, Rb, D), lambda s, r: (s, r, 0)),
            pl.BlockSpec((D, D), c2),
        ],
        out_specs=pl.BlockSpec((1, Rb, D), lambda s, r: (s, r, 0)),
        out_shape=jax.ShapeDtypeStruct((S, N, D), f32),
        compiler_params=pltpu.CompilerParams(
            dimension_semantics=("parallel", "parallel")),
    )(att, xs, wo_t)

    return o.reshape(B, T, N, D)


# speedup vs baseline: 1.1836x; 1.0050x over previous
import functools
import math

import jax
import jax.numpy as jnp
from jax import lax
from jax.experimental import pallas as pl
from jax.experimental.pallas import tpu as pltpu

N_HEADS = 16
KT = 256

_C0 = 6.283203125
_C1 = -1.781781975296326e-05
_C2 = -6.608047442568932e-13
_INV_2PI = 0.15915494309189535


def _reduced_sincos(ang):
    k = jnp.floor(ang * _INV_2PI + 0.5)
    r = ((ang - k * _C0) - k * _C1) - k * _C2
    return jnp.sin(r), jnp.cos(r)


def _rope_emb(p8, dirs_ref, freqs_ref):
    f32 = jnp.float32
    px, py, pz = p8[:, 0:1], p8[:, 1:2], p8[:, 2:3]
    nrm = jnp.sqrt(px * px + py * py + pz * pz)
    inv = 1.0 / jnp.maximum(nrm, 1e-12)
    bf = jnp.bfloat16
    ux = (px * inv).astype(bf).astype(f32)
    uy = (py * inv).astype(bf).astype(f32)
    uz = (pz * inv).astype(bf).astype(f32)
    d0 = dirs_ref[0:1, :].astype(bf).astype(f32)
    d1 = dirs_ref[1:2, :].astype(bf).astype(f32)
    d2 = dirs_ref[2:3, :].astype(bf).astype(f32)
    ang = (ux * d0 + uy * d1 + uz * d2) * freqs_ref[...]
    sin_a, cos_a = _reduced_sincos(ang)
    return jnp.concatenate([sin_a, cos_a], axis=1)


def _compact_kernel(T, padc_ref, spikec_ref, ltri_ref,
                    gx_ref, cnt_ref, biasp_ref):
    f32 = jnp.float32
    N = padc_ref.shape[1]
    s_id = pl.program_id(0)
    sendc = ((spikec_ref[0] != 0) & (padc_ref[0] != 0)).astype(f32)
    cum = jnp.dot(ltri_ref[...], sendc, preferred_element_type=f32)
    n = jnp.sum(sendc, axis=0, keepdims=True)
    jrow = jax.lax.broadcasted_iota(jnp.int32, (1, N), 1).astype(f32)
    le = (cum <= jrow).astype(f32)
    idx = jnp.sum(le, axis=0, keepdims=True)
    idx = jnp.where(n == 0.0, jrow, idx)
    idx = jnp.minimum(idx, f32(N - 1)).astype(jnp.int32)
    gx_ref[0] = idx
    ni = n.astype(jnp.int32)
    cnt_ref[0] = ni
    jcol = jax.lax.broadcasted_iota(jnp.int32, (N, 1), 0)
    biasp_ref[0] = jnp.where(jcol < ni, f32(0), f32(-1e30))


def _prep_kernel(x_ref, pos_ref, rmsw_ref, wq_ref, dirs_ref, freqs_ref, q_ref):
    f32 = jnp.float32
    x = x_ref[0]
    D = x.shape[1]
    H = N_HEADS
    Dh = D // H

    var = jnp.mean(x * x, axis=1, keepdims=True)
    xn = x * jax.lax.rsqrt(var + 1e-6) * rmsw_ref[...]
    emb = _rope_emb(pos_ref[0], dirs_ref, freqs_ref)
    F2 = emb.shape[1]
    qk = jnp.concatenate([xn[:, 0:F2] + emb, xn[:, F2:]], axis=1)

    scale = 1.0 / math.sqrt(Dh)
    q = jnp.dot(qk, wq_ref[...], preferred_element_type=f32) * scale
    for h in range(H):
        q_ref[0, h] = q[:, h * Dh:(h + 1) * Dh]


def _packkv_kernel(idx_ref, cnt_ref, x_ref, pos_ref, rmsw_ref, wk_ref, wv_ref,
                   dirs_ref, freqs_ref, k_ref, v_ref, px_scr, pp_scr):
    f32 = jnp.float32
    N, D = x_ref.shape[1], x_ref.shape[2]
    H = N_HEADS
    Dh = D // H
    n = cnt_ref[0, 0, 0]
    n_eff = jnp.where(n == 0, N, n)
    nt = (n_eff + (KT - 1)) // KT

    def gather_body(j, _):
        i = idx_ref[0, 0, j]
        px_scr[pl.ds(j, 1), :] = x_ref[0, pl.ds(i, 1), :]
        pp_scr[pl.ds(j, 1), :] = pos_ref[0, pl.ds(i, 1), :]
        return 0

    jax.lax.fori_loop(0, n_eff, gather_body, 0)

    def zero_body(j, _):
        px_scr[pl.ds(j, 1), :] = jnp.zeros((1, D), f32)
        pp_scr[pl.ds(j, 1), :] = jnp.zeros((1, 3), f32)
        return 0

    jax.lax.fori_loop(n_eff, nt * KT, zero_body, 0)

    def proj_body(jt, _):
        r0 = jt * KT
        xr = px_scr[pl.ds(r0, KT), :]
        var = jnp.mean(xr * xr, axis=1, keepdims=True)
        xn = xr * jax.lax.rsqrt(var + 1e-6) * rmsw_ref[...]
        emb = _rope_emb(pp_scr[pl.ds(r0, KT), :], dirs_ref, freqs_ref)
        F2 = emb.shape[1]
        rows_k = jnp.concatenate([xn[:, 0:F2] + emb, xn[:, F2:]], axis=1)
        kt = jnp.dot(rows_k, wk_ref[...], preferred_element_type=f32)
        vt = jnp.dot(xn, wv_ref[...], preferred_element_type=f32)
        for h in range(H):
            sl = slice(h * Dh, (h + 1) * Dh)
            k_ref[0, h, pl.ds(r0, KT), :] = kt[:, sl]
            v_ref[0, h, pl.ds(r0, KT), :] = vt[:, sl]
        return 0

    jax.lax.fori_loop(0, nt, proj_body, 0)


def _attn_kernel(cnt_ref, q_ref, k_ref, v_ref, biasp_ref, padc_ref, o_ref):
    f32 = jnp.float32
    HB = q_ref.shape[1]
    N, Dh = q_ref.shape[2], q_ref.shape[3]
    n = cnt_ref[0, 0, 0]
    n_eff = jnp.where(n == 0, N, n)
    nt = (n_eff + (KT - 1)) // KT

    keepc = (padc_ref[0] != 0).astype(f32)
    ones = jnp.ones((N, 1), f32)

    for hh in range(HB):
        q_aug = jnp.concatenate([q_ref[0, hh], ones], axis=1)
        m0 = jnp.full((N, 1), -3e38, f32)
        l0 = jnp.zeros((N, 1), f32)
        acc0 = jnp.zeros((N, Dh), f32)

        def tile_body(jt, carry):
            m, l, acc = carry
            r0 = jt * KT
            k_t = k_ref[0, hh, pl.ds(r0, KT), :]
            v_t = v_ref[0, hh, pl.ds(r0, KT), :]
            b_t = biasp_ref[0, pl.ds(r0, KT), :]
            k_aug = jnp.concatenate([k_t, b_t], axis=1)
            s_t = jax.lax.dot_general(q_aug, k_aug, (((1,), (1,)), ((), ())),
                                      preferred_element_type=f32)
            m_t = jnp.max(s_t, axis=1, keepdims=True)
            m_new = jnp.maximum(m, m_t)
            alpha = jnp.exp(m - m_new)
            e_t = jnp.exp(s_t - m_new)
            l_new = l * alpha + jnp.sum(e_t, axis=1, keepdims=True)
            acc_new = acc * alpha + jnp.dot(e_t.astype(jnp.bfloat16),
                                            v_t.astype(jnp.bfloat16),
                                            preferred_element_type=f32)
            return m_new, l_new, acc_new

        m, l, acc = jax.lax.fori_loop(0, nt, tile_body, (m0, l0, acc0))
        o_ref[0, hh] = acc * (keepc / l)


def _proj_kernel(att_ref, x_ref, wo_ref, o_ref):
    f32 = jnp.float32
    H = att_ref.shape[1]
    out = jnp.concatenate([att_ref[0, h] for h in range(H)], axis=1)
    o_ref[0] = x_ref[0] + jnp.dot(out, wo_ref[...], preferred_element_type=f32)


def kernel(x, point_positions, neuron_pad_mask, spike_mask, rms_w,
           Wq, Wk, Wv, Wo, rope_dirs, rope_freqs):
    B, T, N, D = x.shape
    S = B * T
    H = N_HEADS
    Dh = D // H
    F = rope_dirs.shape[0]
    F2 = 2 * F
    Rb = 256
    R = N // Rb
    f32 = jnp.float32
    i32 = jnp.int32

    xs = x.reshape(S, N, D)
    padc = neuron_pad_mask.reshape(B, N, 1)
    spikec = spike_mask.reshape(S, N, 1)
    rmsw2 = rms_w.reshape(1, D)
    dirs_t = rope_dirs.T
    freqs2 = rope_freqs.reshape(1, F)
    wq_t, wk_t, wv_t, wo_t = Wq.T, Wk.T, Wv.T, Wo.T
    iota_r = jax.lax.broadcasted_iota(f32, (N, N), 0)
    ltri = (jax.lax.broadcasted_iota(f32, (N, N), 1) <= iota_r).astype(f32)

    qkv_shape = jax.ShapeDtypeStruct((S, H, N, Dh), f32)
    c2 = lambda *_: (0, 0)

    sidx, scnt, biasp = pl.pallas_call(
        functools.partial(_compact_kernel, T),
        grid=(S,),
        in_specs=[
            pl.BlockSpec((1, N, 1), lambda s: (s // T, 0, 0)),
            pl.BlockSpec((1, N, 1), lambda s: (s, 0, 0)),
            pl.BlockSpec((N, N), lambda s: (0, 0)),
        ],
        out_specs=[
            pl.BlockSpec((1, 1, N), lambda s: (s, 0, 0)),
            pl.BlockSpec((1, 1, 1), lambda s: (s, 0, 0)),
            pl.BlockSpec((1, N, 1), lambda s: (s, 0, 0)),
        ],
        out_shape=[jax.ShapeDtypeStruct((S, 1, N), i32),
                   jax.ShapeDtypeStruct((S, 1, 1), i32),
                   jax.ShapeDtypeStruct((S, N, 1), f32)],
    )(padc, spikec, ltri)

    q4 = pl.pallas_call(
        _prep_kernel,
        grid=(S, R),
        in_specs=[
            pl.BlockSpec((1, Rb, D), lambda s, r: (s, r, 0)),
            pl.BlockSpec((1, Rb, 3), lambda s, r: (s // T, r, 0)),
            pl.BlockSpec((1, D), c2),
            pl.BlockSpec((D, D), c2),
            pl.BlockSpec((3, F), c2),
            pl.BlockSpec((1, F), c2),
        ],
        out_specs=pl.BlockSpec((1, H, Rb, Dh), lambda s, r: (s, 0, r, 0)),
        out_shape=qkv_shape,
        compiler_params=pltpu.CompilerParams(
            dimension_semantics=("parallel", "parallel")),
    )(xs, point_positions, rmsw2, wq_t, dirs_t, freqs2)

    k4, v4 = pl.pallas_call(
        _packkv_kernel,
        grid=(S,),
        in_specs=[
            pl.BlockSpec(memory_space=pltpu.SMEM,
                         block_shape=(1, 1, N), index_map=lambda s: (s, 0, 0)),
            pl.BlockSpec(memory_space=pltpu.SMEM,
                         block_shape=(1, 1, 1), index_map=lambda s: (s, 0, 0)),
            pl.BlockSpec((1, N, D), lambda s: (s, 0, 0)),
            pl.BlockSpec((1, N, 3), lambda s: (s // T, 0, 0)),
            pl.BlockSpec((1, D), lambda s: (0, 0)),
            pl.BlockSpec((D, D), lambda s: (0, 0)),
            pl.BlockSpec((D, D), lambda s: (0, 0)),
            pl.BlockSpec((3, F), lambda s: (0, 0)),
            pl.BlockSpec((1, F), lambda s: (0, 0)),
        ],
        out_specs=[
            pl.BlockSpec((1, H, N, Dh), lambda s: (s, 0, 0, 0)),
            pl.BlockSpec((1, H, N, Dh), lambda s: (s, 0, 0, 0)),
        ],
        out_shape=[qkv_shape, qkv_shape],
        scratch_shapes=[pltpu.VMEM((N, D), jnp.float32),
                        pltpu.VMEM((N, 3), jnp.float32)],
    )(sidx, scnt, xs, point_positions, rmsw2, wk_t, wv_t, dirs_t, freqs2)

    HB = 4
    att = pl.pallas_call(
        _attn_kernel,
        grid=(S, H // HB),
        in_specs=[
            pl.BlockSpec(memory_space=pltpu.SMEM,
                         block_shape=(1, 1, 1), index_map=lambda s, h: (s, 0, 0)),
            pl.BlockSpec((1, HB, N, Dh), lambda s, h: (s, h, 0, 0)),
            pl.BlockSpec((1, HB, N, Dh), lambda s, h: (s, h, 0, 0)),
            pl.BlockSpec((1, HB, N, Dh), lambda s, h: (s, h, 0, 0)),
            pl.BlockSpec((1, N, 1), lambda s, h: (s, 0, 0)),
            pl.BlockSpec((1, N, 1), lambda s, h: (s // T, 0, 0)),
        ],
        out_specs=pl.BlockSpec((1, HB, N, Dh), lambda s, h: (s, h, 0, 0)),
        out_shape=qkv_shape,
        compiler_params=pltpu.CompilerParams(
            dimension_semantics=("parallel", "parallel")),
    )(scnt, q4, k4, v4, biasp, padc)

    o = pl.pallas_call(
        _proj_kernel,
        grid=(S, R),
        in_specs=[
            pl.BlockSpec((1, H, Rb, Dh), lambda s, r: (s, 0, r, 0)),
            pl.BlockSpec((1, Rb, D), lambda s, r: (s, r, 0)),
            pl.BlockSpec((D, D), c2),
        ],
        out_specs=pl.BlockSpec((1, Rb, D), lambda s, r: (s, r, 0)),
        out_shape=jax.ShapeDtypeStruct((S, N, D), f32),
        compiler_params=pltpu.CompilerParams(
            dimension_semantics=("parallel", "parallel")),
    )(att, xs, wo_t)

    return o.reshape(B, T, N, D)
